# Initial kernel scaffold; baseline (speedup 1.0000x reference)
import jax
import jax.numpy as jnp
from jax.experimental import pallas as pl


def kernel(x, edge_index, Wl0, Wr0, b0, g0, be0, Wl1, Wr1, b1, g1, be1, cW1, cb1, cW2, cb2):
    raise NotImplementedError("write your pallas kernel here")


# speedup vs baseline: 5.9094x; 5.9094x over previous
import functools

import jax
import jax.numpy as jnp
from jax import lax
from jax.experimental import pallas as pl
from jax.experimental.pallas import tpu as pltpu
from jax.experimental.pallas import tpu_sc as plsc

_N = 10000
_E = 320000
_H = 128

_NCORE = 2
_NSUB = 16
_NW = _NCORE * _NSUB

_NP = 10240
_CHUNK = _NP // _NSUB
_B = 128
_NBATCH = 79
_EPW = _B * _NBATCH
_EPAD = _EPW * _NW
_DUMMY = _N

_R = 1024
_G = 10


_sc_mesh = plsc.VectorSubcoreMesh(core_axis_name="c", subcore_axis_name="s")


@functools.partial(
    pl.kernel,
    out_type=[
        jax.ShapeDtypeStruct((_NCORE, _NP, _H), jnp.float32),
        jax.ShapeDtypeStruct((_NCORE, _NP), jnp.float32),
    ],
    mesh=_sc_mesh,
    scratch_types=[
        pltpu.VMEM((_B,), jnp.int32),
        pltpu.VMEM((_B,), jnp.int32),
        pltpu.VMEM((_B, _H), jnp.float32),
        pltpu.VMEM((_B,), jnp.float32),
        pltpu.VMEM_SHARED((_NP, _H), jnp.float32),
        pltpu.VMEM_SHARED((_NP,), jnp.float32),
        pltpu.SemaphoreType.DMA,
    ],
)
def _sc_agg(feat_hbm, src_hbm, dst_hbm, zrows_hbm, zvec_hbm, ones_hbm,
            psum_hbm, pdeg_hbm,
            src_v, dst_v, rows_v, ones_v, acc_sh, deg_sh, sem):
    c = lax.axis_index("c")
    s = lax.axis_index("s")
    wid = s * _NCORE + c
    row0 = s * _CHUNK

    for k in range(_CHUNK // _B):
        pltpu.sync_copy(zrows_hbm, acc_sh.at[pl.ds(row0 + k * _B, _B)])
        pltpu.sync_copy(zvec_hbm, deg_sh.at[pl.ds(row0 + k * _B, _B)])
    pltpu.sync_copy(ones_hbm, ones_v)
    plsc.subcore_barrier()

    ebase = wid * _EPW

    def body(b, carry):
        off = ebase + b * _B
        pltpu.sync_copy(src_hbm.at[pl.ds(off, _B)], src_v)
        pltpu.sync_copy(dst_hbm.at[pl.ds(off, _B)], dst_v)
        pltpu.async_copy(feat_hbm.at[src_v], rows_v, sem).wait()
        pltpu.sync_copy(rows_v, acc_sh.at[dst_v], add=True)
        pltpu.sync_copy(ones_v, deg_sh.at[dst_v], add=True)
        return carry

    lax.fori_loop(0, _NBATCH, body, 0)
    plsc.subcore_barrier()

    pltpu.sync_copy(acc_sh.at[pl.ds(row0, _CHUNK)],
                    psum_hbm.at[c, pl.ds(row0, _CHUNK)])
    pltpu.sync_copy(deg_sh.at[pl.ds(row0, _CHUNK)],
                    pdeg_hbm.at[c, pl.ds(row0, _CHUNK)])


def _layer_block(p_ref, dg_ref, x_ref, wl_ref, wr_ref, b_ref, g_ref, be_ref):
    psum = p_ref[0] + p_ref[1]
    deg = dg_ref[0] + dg_ref[1]
    mean = psum / jnp.maximum(deg, 1.0)
    h = jnp.dot(mean, wl_ref[...], preferred_element_type=jnp.float32)
    h = h + jnp.dot(x_ref[...], wr_ref[...], preferred_element_type=jnp.float32)
    h = h + b_ref[...]
    mu = jnp.mean(h, axis=-1, keepdims=True)
    var = jnp.mean((h - mu) * (h - mu), axis=-1, keepdims=True)
    h = (h - mu) * lax.rsqrt(var + 1e-5) * g_ref[...] + be_ref[...]
    return jnp.maximum(h, 0.0)


def _dense0_body(p_ref, dg_ref, x_ref, wl_ref, wr_ref, b_ref, g_ref, be_ref,
                 o_ref):
    o_ref[...] = _layer_block(p_ref, dg_ref, x_ref, wl_ref, wr_ref, b_ref,
                              g_ref, be_ref)


def _dense1_body(p_ref, dg_ref, h_ref, wl_ref, wr_ref, b_ref, g_ref, be_ref,
                 cw1_ref, cb1_ref, cw2_ref, cb2_ref, o_ref):
    h1 = _layer_block(p_ref, dg_ref, h_ref, wl_ref, wr_ref, b_ref, g_ref,
                      be_ref)
    t = jnp.dot(h1, cw1_ref[...], preferred_element_type=jnp.float32)
    t = jnp.maximum(t + cb1_ref[...], 0.0)
    o_ref[...] = (jnp.dot(t, cw2_ref[...], preferred_element_type=jnp.float32)
                  + cb2_ref[...])


_full = pl.BlockSpec((_H, _H), lambda i: (0, 0))
_brow = pl.BlockSpec((1, _H), lambda i: (0, 0))
_pspec = pl.BlockSpec((_NCORE, _R, _H), lambda i: (0, i, 0))
_dgspec = pl.BlockSpec((_NCORE, _R, 1), lambda i: (0, i, 0))
_rowspec = pl.BlockSpec((_R, _H), lambda i: (i, 0))

_dense0 = pl.pallas_call(
    _dense0_body,
    grid=(_G,),
    in_specs=[_pspec, _dgspec, _rowspec, _full, _full, _brow, _brow, _brow],
    out_specs=_rowspec,
    out_shape=jax.ShapeDtypeStruct((_N, _H), jnp.float32),
)

_dense1 = pl.pallas_call(
    _dense1_body,
    grid=(_G,),
    in_specs=[_pspec, _dgspec, _rowspec, _full, _full, _brow, _brow, _brow,
              _full, _brow, pl.BlockSpec((_H, 1), lambda i: (0, 0)),
              pl.BlockSpec((1, 1), lambda i: (0, 0))],
    out_specs=pl.BlockSpec((_R, 1), lambda i: (i, 0)),
    out_shape=jax.ShapeDtypeStruct((_N, 1), jnp.float32),
)


def kernel(x, edge_index, Wl0, Wr0, b0, g0, be0, Wl1, Wr1, b1, g1, be1,
           cW1, cb1, cW2, cb2):
    npad = _EPAD - _E
    pad_iota = lax.iota(jnp.int32, npad)
    src = jnp.concatenate([edge_index[0], pad_iota % _N])
    dst = jnp.concatenate([edge_index[1], _DUMMY + pad_iota % (_NP - _N)])
    zrows = jnp.zeros((_B, _H), jnp.float32)
    zvec = jnp.zeros((_B,), jnp.float32)
    ones = jnp.ones((_B,), jnp.float32)

    p0, dg = _sc_agg(x, src, dst, zrows, zvec, ones)
    dg3 = dg.reshape(_NCORE, _NP, 1)
    h0 = _dense0(p0, dg3, x, Wl0, Wr0, b0.reshape(1, _H), g0.reshape(1, _H),
                 be0.reshape(1, _H))
    p1, _ = _sc_agg(h0, src, dst, zrows, zvec, ones)
    out = _dense1(p1, dg3, h0, Wl1, Wr1, b1.reshape(1, _H),
                  g1.reshape(1, _H), be1.reshape(1, _H),
                  cW1, cb1.reshape(1, _H), cW2, cb2.reshape(1, 1))
    return out


# speedup vs baseline: 10.3834x; 1.7571x over previous
import functools

import jax
import jax.numpy as jnp
from jax import lax
from jax.experimental import pallas as pl
from jax.experimental.pallas import tpu as pltpu
from jax.experimental.pallas import tpu_sc as plsc

_N = 10000
_E = 320000
_H = 128

_NCORE = 2
_NSUB = 16
_NW = _NCORE * _NSUB

_NP = 10240
_CHUNK = _NP // _NSUB
_B = 128
_NBATCH = 80
_NBUF = 4
_EPW = _B * _NBATCH
_EPAD = _EPW * _NW
_DUMMY = _N

_R = 1024
_G = 10


_sc_mesh = plsc.VectorSubcoreMesh(core_axis_name="c", subcore_axis_name="s")


def _make_sc_agg(compute_deg):
    out_type = [jax.ShapeDtypeStruct((_NCORE, _NP, _H), jnp.float32)]
    scratch = [
        pltpu.VMEM_SHARED((_NP, _H), jnp.float32),
        pltpu.VMEM((2, _B), jnp.int32),
        pltpu.VMEM((2, _B), jnp.int32),
        pltpu.VMEM((_B, _H), jnp.float32),
        pltpu.VMEM((_B, _H), jnp.float32),
        pltpu.SemaphoreType.DMA,
        pltpu.SemaphoreType.DMA,
    ]
    if compute_deg:
        out_type.append(jax.ShapeDtypeStruct((_NCORE, _NP), jnp.float32))
        scratch += [
            pltpu.VMEM((_B,), jnp.float32),
            pltpu.VMEM_SHARED((_NP,), jnp.float32),
        ]

    @functools.partial(pl.kernel, out_type=out_type, mesh=_sc_mesh,
                       scratch_types=scratch)
    def sc_agg(feat_hbm, idx_hbm, zrows_hbm, zvec_hbm, ones_hbm, *rest):
        if compute_deg:
            psum_hbm, pdeg_hbm, acc_sh, i0, i1, r0, r1, s0, s1, ones_v, \
                deg_sh = rest
        else:
            psum_hbm, acc_sh, i0, i1, r0, r1, s0, s1 = rest

        c = lax.axis_index("c")
        s = lax.axis_index("s")
        wid = s * _NCORE + c
        row0 = s * _CHUNK
        ibase = wid * _NBATCH

        pltpu.sync_copy(zrows_hbm.at[pl.ds(row0, _CHUNK)],
                        acc_sh.at[pl.ds(row0, _CHUNK)])
        if compute_deg:
            pltpu.sync_copy(zvec_hbm.at[pl.ds(row0, _CHUNK)],
                            deg_sh.at[pl.ds(row0, _CHUNK)])
            pltpu.sync_copy(ones_hbm, ones_v)
        plsc.subcore_barrier()

        def scatter(ibuf, rbuf):
            pltpu.sync_copy(rbuf, acc_sh.at[ibuf.at[1]], add=True)
            if compute_deg:
                pltpu.sync_copy(ones_v, deg_sh.at[ibuf.at[1]], add=True)

        pltpu.sync_copy(idx_hbm.at[ibase], i0)
        pltpu.async_copy(feat_hbm.at[i0.at[0]], r0, s0)
        pltpu.sync_copy(idx_hbm.at[ibase + 1], i1)

        def pair(i, carry):
            j0 = 2 * i
            pltpu.make_async_copy(feat_hbm.at[i0.at[0]], r0, s0).wait()
            pltpu.async_copy(feat_hbm.at[i1.at[0]], r1, s1)
            scatter(i0, r0)

            @pl.when(j0 + 2 < _NBATCH)
            def _():
                pltpu.sync_copy(idx_hbm.at[ibase + j0 + 2], i0)

            pltpu.make_async_copy(feat_hbm.at[i1.at[0]], r1, s1).wait()

            @pl.when(j0 + 2 < _NBATCH)
            def _():
                pltpu.async_copy(feat_hbm.at[i0.at[0]], r0, s0)

            scatter(i1, r1)

            @pl.when(j0 + 3 < _NBATCH)
            def _():
                pltpu.sync_copy(idx_hbm.at[ibase + j0 + 3], i1)

            return carry

        lax.fori_loop(0, _NBATCH // 2, pair, 0)
        plsc.subcore_barrier()

        pltpu.sync_copy(acc_sh.at[pl.ds(row0, _CHUNK)],
                        psum_hbm.at[c, pl.ds(row0, _CHUNK)])
        if compute_deg:
            pltpu.sync_copy(deg_sh.at[pl.ds(row0, _CHUNK)],
                            pdeg_hbm.at[c, pl.ds(row0, _CHUNK)])

    return sc_agg


_sc_agg_deg = _make_sc_agg(True)
_sc_agg_nodeg = _make_sc_agg(False)


def _layer_block(p_ref, dg_ref, x_ref, wl_ref, wr_ref, b_ref, g_ref, be_ref):
    psum = p_ref[0] + p_ref[1]
    deg = dg_ref[0] + dg_ref[1]
    mean = psum / jnp.maximum(deg, 1.0)
    h = jnp.dot(mean, wl_ref[...], preferred_element_type=jnp.float32)
    h = h + jnp.dot(x_ref[...], wr_ref[...], preferred_element_type=jnp.float32)
    h = h + b_ref[...]
    mu = jnp.mean(h, axis=-1, keepdims=True)
    var = jnp.mean((h - mu) * (h - mu), axis=-1, keepdims=True)
    h = (h - mu) * lax.rsqrt(var + 1e-5) * g_ref[...] + be_ref[...]
    return jnp.maximum(h, 0.0)


def _dense0_body(p_ref, dg_ref, x_ref, wl_ref, wr_ref, b_ref, g_ref, be_ref,
                 o_ref):
    o_ref[...] = _layer_block(p_ref, dg_ref, x_ref, wl_ref, wr_ref, b_ref,
                              g_ref, be_ref)


def _dense1_body(p_ref, dg_ref, h_ref, wl_ref, wr_ref, b_ref, g_ref, be_ref,
                 cw1_ref, cb1_ref, cw2_ref, cb2_ref, o_ref):
    h1 = _layer_block(p_ref, dg_ref, h_ref, wl_ref, wr_ref, b_ref, g_ref,
                      be_ref)
    t = jnp.dot(h1, cw1_ref[...], preferred_element_type=jnp.float32)
    t = jnp.maximum(t + cb1_ref[...], 0.0)
    o_ref[...] = (jnp.dot(t, cw2_ref[...], preferred_element_type=jnp.float32)
                  + cb2_ref[...])


_full = pl.BlockSpec((_H, _H), lambda i: (0, 0))
_brow = pl.BlockSpec((1, _H), lambda i: (0, 0))
_pspec = pl.BlockSpec((_NCORE, _R, _H), lambda i: (0, i, 0))
_dgspec = pl.BlockSpec((_NCORE, _R, 1), lambda i: (0, i, 0))
_rowspec = pl.BlockSpec((_R, _H), lambda i: (i, 0))

_dense0 = pl.pallas_call(
    _dense0_body,
    grid=(_G,),
    in_specs=[_pspec, _dgspec, _rowspec, _full, _full, _brow, _brow, _brow],
    out_specs=_rowspec,
    out_shape=jax.ShapeDtypeStruct((_N, _H), jnp.float32),
)

_dense1 = pl.pallas_call(
    _dense1_body,
    grid=(_G,),
    in_specs=[_pspec, _dgspec, _rowspec, _full, _full, _brow, _brow, _brow,
              _full, _brow, pl.BlockSpec((_H, 1), lambda i: (0, 0)),
              pl.BlockSpec((1, 1), lambda i: (0, 0))],
    out_specs=pl.BlockSpec((_R, 1), lambda i: (i, 0)),
    out_shape=jax.ShapeDtypeStruct((_N, 1), jnp.float32),
)


def kernel(x, edge_index, Wl0, Wr0, b0, g0, be0, Wl1, Wr1, b1, g1, be1,
           cW1, cb1, cW2, cb2):
    npad = _EPAD - _E
    pad_iota = lax.iota(jnp.int32, npad)
    src = jnp.concatenate([edge_index[0], pad_iota % _N])
    dst = jnp.concatenate([edge_index[1], _DUMMY + pad_iota % (_NP - _N)])
    idx = jnp.stack([src.reshape(_NW * _NBATCH, _B),
                     dst.reshape(_NW * _NBATCH, _B)], axis=1)
    zrows = jnp.zeros((_NP, _H), jnp.float32)
    zvec = jnp.zeros((_NP,), jnp.float32)
    ones = jnp.ones((_B,), jnp.float32)

    p0, dg = _sc_agg_deg(x, idx, zrows, zvec, ones)
    dg3 = dg.reshape(_NCORE, _NP, 1)
    h0 = _dense0(p0, dg3, x, Wl0, Wr0, b0.reshape(1, _H), g0.reshape(1, _H),
                 be0.reshape(1, _H))
    (p1,) = _sc_agg_nodeg(h0, idx, zrows, zvec, ones)
    out = _dense1(p1, dg3, h0, Wl1, Wr1, b1.reshape(1, _H),
                  g1.reshape(1, _H), be1.reshape(1, _H),
                  cW1, cb1.reshape(1, _H), cW2, cb2.reshape(1, 1))
    return out
